# baseline (speedup 1.0000x reference)
import functools

import jax
import jax.numpy as jnp
from jax.experimental import pallas as pl
from jax.experimental.pallas import tpu as pltpu

N = 512
D = 64
MSG = 64
F_HID = 128
K = 8
NC = N // K


def _bf(v):
    return v.astype(jnp.bfloat16)


def _sum_body(hid_ref, wf1a_ref, wf1b_ref, bf1_ref, wf2_ref, adj3_ref,
              s_ref, lhst_s, a2t_s):
    c = pl.program_id(0)

    @pl.when(c == 0)
    def _():
        hidt16 = _bf(hid_ref[:].T)
        t_idx = jax.lax.broadcasted_iota(jnp.int32, (2 * K, K * N), 0)
        k_idx = jax.lax.broadcasted_iota(jnp.int32, (2 * K, K * N), 1) // N
        lhst_s[:2 * K, :] = (t_idx % K == k_idx).astype(jnp.float32).astype(jnp.bfloat16)
        for k in range(K):
            lhst_s[2 * K:, k * N:(k + 1) * N] = hidt16
        a = (jnp.dot(wf1a_ref[:], hidt16, preferred_element_type=jnp.float32)
             + bf1_ref[:])
        ahi = _bf(a)
        alo = _bf(a - ahi.astype(jnp.float32))
        for cc in range(NC):
            a2t_s[cc, :, :K] = ahi[:, cc * K:(cc + 1) * K]
            a2t_s[cc, :, K:] = alo[:, cc * K:(cc + 1) * K]
        s_ref[:] = jnp.zeros_like(s_ref)

    rhst = jnp.concatenate([a2t_s[c], wf1b_ref[:]], axis=1)
    pre = jnp.dot(rhst, lhst_s[:], preferred_element_type=jnp.float32)
    relu = jnp.maximum(pre, 0.0)
    msgs = jnp.dot(wf2_ref[:], relu, preferred_element_type=jnp.float32)
    m = adj3_ref[0].astype(jnp.float32)
    terms = [m[k:k + 1, :] * msgs[:, k * N:(k + 1) * N] for k in range(K)]
    while len(terms) > 1:
        terms = [terms[i] + terms[i + 1] for i in range(0, len(terms), 2)]
    s_ref[:] += terms[0]


def _tail_body(st_ref, adj_ref, x_ref, hprev_ref, cprev_ref,
               bf2_ref, wih_ref, whh_ref, bsum_ref,
               wo1_ref, bo1_ref, wo2_ref, bo2_ref,
               out_ref, h_ref, c_ref):
    deg = jnp.sum(adj_ref[:], axis=0, keepdims=True).astype(jnp.float32)
    msgt = st_ref[:] + deg * bf2_ref[:]

    inpt = jnp.concatenate([x_ref[:].T, msgt], axis=0).astype(jnp.bfloat16)
    gates = (jnp.dot(wih_ref[:], inpt, preferred_element_type=jnp.float32)
             + jnp.dot(whh_ref[:], _bf(hprev_ref[0].T),
                       preferred_element_type=jnp.float32)
             + bsum_ref[:])
    i_g = jax.nn.sigmoid(gates[0 * D:1 * D, :])
    f_g = jax.nn.sigmoid(gates[1 * D:2 * D, :])
    g_g = jnp.tanh(gates[2 * D:3 * D, :])
    o_g = jax.nn.sigmoid(gates[3 * D:4 * D, :])
    c_new = f_g * cprev_ref[0].T + i_g * g_g
    h_new = o_g * jnp.tanh(c_new)

    hid1 = jnp.maximum(
        jnp.dot(wo1_ref[:], h_new.astype(jnp.bfloat16),
                preferred_element_type=jnp.float32) + bo1_ref[:], 0.0)
    out = (jnp.dot(wo2_ref[:], hid1.astype(jnp.bfloat16),
                   preferred_element_type=jnp.float32) + bo2_ref[:])
    out_ref[:] = out.T
    h_ref[:] = h_new.T
    c_ref[:] = c_new.T


@functools.partial(jax.jit, static_argnames=("interpret",))
def _run(adjacency_matrix, x, hidden, h_h, h_c, Wf1, bf1, Wf2, bf2,
         W_ih, W_hh, b_ih, b_hh, Wo1, bo1, Wo2, bo2, interpret=False):
    n, d = hidden.shape
    adj3 = adjacency_matrix.reshape(NC, K, n)

    St = pl.pallas_call(
        _sum_body,
        grid=(NC,),
        in_specs=[
            pl.BlockSpec((n, d), lambda c: (0, 0)),
            pl.BlockSpec((F_HID, d), lambda c: (0, 0)),
            pl.BlockSpec((F_HID, d), lambda c: (0, 0)),
            pl.BlockSpec((F_HID, 1), lambda c: (0, 0)),
            pl.BlockSpec((MSG, F_HID), lambda c: (0, 0)),
            pl.BlockSpec((1, K, n), lambda c: (c, 0, 0)),
        ],
        out_specs=pl.BlockSpec((MSG, n), lambda c: (0, 0)),
        out_shape=jax.ShapeDtypeStruct((MSG, n), jnp.float32),
        scratch_shapes=[
            pltpu.VMEM((2 * K + d, K * n), jnp.bfloat16),
            pltpu.VMEM((NC, F_HID, 2 * K), jnp.bfloat16),
        ],
        interpret=interpret,
    )(hidden, _bf(Wf1[:, :d]), _bf(Wf1[:, d:]), bf1[:, None],
      _bf(Wf2).astype(jnp.float32), adj3)

    out, h_new, c_new = pl.pallas_call(
        _tail_body,
        out_shape=[
            jax.ShapeDtypeStruct((n, Wo2.shape[0]), jnp.float32),
            jax.ShapeDtypeStruct((n, d), jnp.float32),
            jax.ShapeDtypeStruct((n, d), jnp.float32),
        ],
        interpret=interpret,
    )(St, adjacency_matrix, x, h_h, h_c,
      bf2[:, None], _bf(W_ih), _bf(W_hh), (b_ih + b_hh)[:, None],
      _bf(Wo1), bo1[:, None], _bf(Wo2), bo2[:, None])
    return out, h_new, h_new[None, :, :], c_new[None, :, :]


def kernel(adjacency_matrix, x, hidden, h_h, h_c, Wf1, bf1, Wf2, bf2,
           W_ih, W_hh, b_ih, b_hh, Wo1, bo1, Wo2, bo2):
    return _run(adjacency_matrix, x, hidden, h_h, h_c, Wf1, bf1, Wf2, bf2,
                W_ih, W_hh, b_ih, b_hh, Wo1, bo1, Wo2, bo2)


# speedup vs baseline: 1.1984x; 1.1984x over previous
import functools

import jax
import jax.numpy as jnp
from jax.experimental import pallas as pl
from jax.experimental.pallas import tpu as pltpu

N = 512
D = 64
MSG = 64
F_HID = 128
K = 16
NC = N // K


def _bf(v):
    return v.astype(jnp.bfloat16)


def _sum_body(hid_ref, wf1a_ref, wf1b_ref, bf1_ref, wf2_ref, adj3_ref,
              s_ref, lhst_s, a2t_s):
    c = pl.program_id(0)

    @pl.when(c == 0)
    def _():
        hidt16 = _bf(hid_ref[:].T)
        t_idx = jax.lax.broadcasted_iota(jnp.int32, (2 * K, K * N), 0)
        k_idx = jax.lax.broadcasted_iota(jnp.int32, (2 * K, K * N), 1) // N
        lhst_s[:2 * K, :] = (t_idx % K == k_idx).astype(jnp.float32).astype(jnp.bfloat16)
        for k in range(K):
            lhst_s[2 * K:, k * N:(k + 1) * N] = hidt16
        a = (jnp.dot(wf1a_ref[:], hidt16, preferred_element_type=jnp.float32)
             + bf1_ref[:])
        ahi = _bf(a)
        alo = _bf(a - ahi.astype(jnp.float32))
        for cc in range(NC):
            a2t_s[cc, :, :K] = ahi[:, cc * K:(cc + 1) * K]
            a2t_s[cc, :, K:] = alo[:, cc * K:(cc + 1) * K]
        s_ref[:] = jnp.zeros_like(s_ref)

    rhst = jnp.concatenate([a2t_s[c], wf1b_ref[:]], axis=1)
    pre = jnp.dot(rhst, lhst_s[:], preferred_element_type=jnp.float32)
    relu = jnp.maximum(pre, 0.0)
    msgs = jnp.dot(wf2_ref[:], relu, preferred_element_type=jnp.float32)
    m = adj3_ref[0].astype(jnp.float32)
    terms = [m[k:k + 1, :] * msgs[:, k * N:(k + 1) * N] for k in range(K)]
    while len(terms) > 1:
        terms = [terms[i] + terms[i + 1] for i in range(0, len(terms), 2)]
    s_ref[:] += terms[0]


def _tail_body(st_ref, adj_ref, x_ref, hprev_ref, cprev_ref,
               bf2_ref, wih_ref, whh_ref, bsum_ref,
               wo1_ref, bo1_ref, wo2_ref, bo2_ref,
               out_ref, h_ref, c_ref):
    deg = jnp.sum(adj_ref[:], axis=0, keepdims=True).astype(jnp.float32)
    msgt = st_ref[:] + deg * bf2_ref[:]

    inpt = jnp.concatenate([x_ref[:].T, msgt], axis=0).astype(jnp.bfloat16)
    gates = (jnp.dot(wih_ref[:], inpt, preferred_element_type=jnp.float32)
             + jnp.dot(whh_ref[:], _bf(hprev_ref[0].T),
                       preferred_element_type=jnp.float32)
             + bsum_ref[:])
    i_g = jax.nn.sigmoid(gates[0 * D:1 * D, :])
    f_g = jax.nn.sigmoid(gates[1 * D:2 * D, :])
    g_g = jnp.tanh(gates[2 * D:3 * D, :])
    o_g = jax.nn.sigmoid(gates[3 * D:4 * D, :])
    c_new = f_g * cprev_ref[0].T + i_g * g_g
    h_new = o_g * jnp.tanh(c_new)

    hid1 = jnp.maximum(
        jnp.dot(wo1_ref[:], h_new.astype(jnp.bfloat16),
                preferred_element_type=jnp.float32) + bo1_ref[:], 0.0)
    out = (jnp.dot(wo2_ref[:], hid1.astype(jnp.bfloat16),
                   preferred_element_type=jnp.float32) + bo2_ref[:])
    out_ref[:] = out.T
    h_ref[:] = h_new.T
    c_ref[:] = c_new.T


@functools.partial(jax.jit, static_argnames=("interpret",))
def _run(adjacency_matrix, x, hidden, h_h, h_c, Wf1, bf1, Wf2, bf2,
         W_ih, W_hh, b_ih, b_hh, Wo1, bo1, Wo2, bo2, interpret=False):
    n, d = hidden.shape
    adj3 = adjacency_matrix.reshape(NC, K, n)

    St = pl.pallas_call(
        _sum_body,
        grid=(NC,),
        in_specs=[
            pl.BlockSpec((n, d), lambda c: (0, 0)),
            pl.BlockSpec((F_HID, d), lambda c: (0, 0)),
            pl.BlockSpec((F_HID, d), lambda c: (0, 0)),
            pl.BlockSpec((F_HID, 1), lambda c: (0, 0)),
            pl.BlockSpec((MSG, F_HID), lambda c: (0, 0)),
            pl.BlockSpec((1, K, n), lambda c: (c, 0, 0)),
        ],
        out_specs=pl.BlockSpec((MSG, n), lambda c: (0, 0)),
        out_shape=jax.ShapeDtypeStruct((MSG, n), jnp.float32),
        scratch_shapes=[
            pltpu.VMEM((2 * K + d, K * n), jnp.bfloat16),
            pltpu.VMEM((NC, F_HID, 2 * K), jnp.bfloat16),
        ],
        interpret=interpret,
    )(hidden, _bf(Wf1[:, :d]), _bf(Wf1[:, d:]), bf1[:, None],
      _bf(Wf2).astype(jnp.float32), adj3)

    out, h_new, c_new = pl.pallas_call(
        _tail_body,
        out_shape=[
            jax.ShapeDtypeStruct((n, Wo2.shape[0]), jnp.float32),
            jax.ShapeDtypeStruct((n, d), jnp.float32),
            jax.ShapeDtypeStruct((n, d), jnp.float32),
        ],
        interpret=interpret,
    )(St, adjacency_matrix, x, h_h, h_c,
      bf2[:, None], _bf(W_ih), _bf(W_hh), (b_ih + b_hh)[:, None],
      _bf(Wo1), bo1[:, None], _bf(Wo2), bo2[:, None])
    return out, h_new, h_new[None, :, :], c_new[None, :, :]


def kernel(adjacency_matrix, x, hidden, h_h, h_c, Wf1, bf1, Wf2, bf2,
           W_ih, W_hh, b_ih, b_hh, Wo1, bo1, Wo2, bo2):
    return _run(adjacency_matrix, x, hidden, h_h, h_c, Wf1, bf1, Wf2, bf2,
                W_ih, W_hh, b_ih, b_hh, Wo1, bo1, Wo2, bo2)


# speedup vs baseline: 1.2964x; 1.0817x over previous
import functools

import jax
import jax.numpy as jnp
from jax.experimental import pallas as pl
from jax.experimental.pallas import tpu as pltpu

N = 512
D = 64
MSG = 64
F_HID = 128
K = 32
NC = N // K


def _bf(v):
    return v.astype(jnp.bfloat16)


def _sum_body(hid_ref, wf1a_ref, wf1b_ref, bf1_ref, wf2_ref, adj3_ref,
              s_ref, lhst_s, a2t_s):
    c = pl.program_id(0)

    @pl.when(c == 0)
    def _():
        hidt16 = _bf(hid_ref[:].T)
        t_idx = jax.lax.broadcasted_iota(jnp.int32, (2 * K, K * N), 0)
        k_idx = jax.lax.broadcasted_iota(jnp.int32, (2 * K, K * N), 1) // N
        lhst_s[:2 * K, :] = (t_idx % K == k_idx).astype(jnp.float32).astype(jnp.bfloat16)
        for k in range(K):
            lhst_s[2 * K:, k * N:(k + 1) * N] = hidt16
        a = (jnp.dot(wf1a_ref[:], hidt16, preferred_element_type=jnp.float32)
             + bf1_ref[:])
        ahi = _bf(a)
        alo = _bf(a - ahi.astype(jnp.float32))
        for cc in range(NC):
            a2t_s[cc, :, :K] = ahi[:, cc * K:(cc + 1) * K]
            a2t_s[cc, :, K:] = alo[:, cc * K:(cc + 1) * K]
        s_ref[:] = jnp.zeros_like(s_ref)

    rhst = jnp.concatenate([a2t_s[c], wf1b_ref[:]], axis=1)
    pre = jnp.dot(rhst, lhst_s[:], preferred_element_type=jnp.float32)
    relu = jnp.maximum(pre, 0.0)
    msgs = jnp.dot(wf2_ref[:], relu, preferred_element_type=jnp.float32)
    m = adj3_ref[0].astype(jnp.float32)
    terms = [m[k:k + 1, :] * msgs[:, k * N:(k + 1) * N] for k in range(K)]
    while len(terms) > 1:
        terms = [terms[i] + terms[i + 1] for i in range(0, len(terms), 2)]
    s_ref[:] += terms[0]


def _tail_body(st_ref, adj_ref, x_ref, hprev_ref, cprev_ref,
               bf2_ref, wih_ref, whh_ref, bsum_ref,
               wo1_ref, bo1_ref, wo2_ref, bo2_ref,
               out_ref, h_ref, c_ref):
    deg = jnp.sum(adj_ref[:], axis=0, keepdims=True).astype(jnp.float32)
    msgt = st_ref[:] + deg * bf2_ref[:]

    inpt = jnp.concatenate([x_ref[:].T, msgt], axis=0).astype(jnp.bfloat16)
    gates = (jnp.dot(wih_ref[:], inpt, preferred_element_type=jnp.float32)
             + jnp.dot(whh_ref[:], _bf(hprev_ref[0].T),
                       preferred_element_type=jnp.float32)
             + bsum_ref[:])
    i_g = jax.nn.sigmoid(gates[0 * D:1 * D, :])
    f_g = jax.nn.sigmoid(gates[1 * D:2 * D, :])
    g_g = jnp.tanh(gates[2 * D:3 * D, :])
    o_g = jax.nn.sigmoid(gates[3 * D:4 * D, :])
    c_new = f_g * cprev_ref[0].T + i_g * g_g
    h_new = o_g * jnp.tanh(c_new)

    hid1 = jnp.maximum(
        jnp.dot(wo1_ref[:], h_new.astype(jnp.bfloat16),
                preferred_element_type=jnp.float32) + bo1_ref[:], 0.0)
    out = (jnp.dot(wo2_ref[:], hid1.astype(jnp.bfloat16),
                   preferred_element_type=jnp.float32) + bo2_ref[:])
    out_ref[:] = out.T
    h_ref[:] = h_new.T
    c_ref[:] = c_new.T


@functools.partial(jax.jit, static_argnames=("interpret",))
def _run(adjacency_matrix, x, hidden, h_h, h_c, Wf1, bf1, Wf2, bf2,
         W_ih, W_hh, b_ih, b_hh, Wo1, bo1, Wo2, bo2, interpret=False):
    n, d = hidden.shape
    adj3 = adjacency_matrix.reshape(NC, K, n)

    St = pl.pallas_call(
        _sum_body,
        grid=(NC,),
        in_specs=[
            pl.BlockSpec((n, d), lambda c: (0, 0)),
            pl.BlockSpec((F_HID, d), lambda c: (0, 0)),
            pl.BlockSpec((F_HID, d), lambda c: (0, 0)),
            pl.BlockSpec((F_HID, 1), lambda c: (0, 0)),
            pl.BlockSpec((MSG, F_HID), lambda c: (0, 0)),
            pl.BlockSpec((1, K, n), lambda c: (c, 0, 0)),
        ],
        out_specs=pl.BlockSpec((MSG, n), lambda c: (0, 0)),
        out_shape=jax.ShapeDtypeStruct((MSG, n), jnp.float32),
        scratch_shapes=[
            pltpu.VMEM((2 * K + d, K * n), jnp.bfloat16),
            pltpu.VMEM((NC, F_HID, 2 * K), jnp.bfloat16),
        ],
        interpret=interpret,
    )(hidden, _bf(Wf1[:, :d]), _bf(Wf1[:, d:]), bf1[:, None],
      _bf(Wf2).astype(jnp.float32), adj3)

    out, h_new, c_new = pl.pallas_call(
        _tail_body,
        out_shape=[
            jax.ShapeDtypeStruct((n, Wo2.shape[0]), jnp.float32),
            jax.ShapeDtypeStruct((n, d), jnp.float32),
            jax.ShapeDtypeStruct((n, d), jnp.float32),
        ],
        interpret=interpret,
    )(St, adjacency_matrix, x, h_h, h_c,
      bf2[:, None], _bf(W_ih), _bf(W_hh), (b_ih + b_hh)[:, None],
      _bf(Wo1), bo1[:, None], _bf(Wo2), bo2[:, None])
    return out, h_new, h_new[None, :, :], c_new[None, :, :]


def kernel(adjacency_matrix, x, hidden, h_h, h_c, Wf1, bf1, Wf2, bf2,
           W_ih, W_hh, b_ih, b_hh, Wo1, bo1, Wo2, bo2):
    return _run(adjacency_matrix, x, hidden, h_h, h_c, Wf1, bf1, Wf2, bf2,
                W_ih, W_hh, b_ih, b_hh, Wo1, bo1, Wo2, bo2)


# speedup vs baseline: 1.3494x; 1.0409x over previous
import functools

import jax
import jax.numpy as jnp
from jax.experimental import pallas as pl
from jax.experimental.pallas import tpu as pltpu

N = 512
D = 64
MSG = 64
F_HID = 128
K = 64
NC = N // K


def _bf(v):
    return v.astype(jnp.bfloat16)


def _sum_body(hid_ref, wf1a_ref, wf1b_ref, bf1_ref, wf2_ref, adj3_ref,
              s_ref, lhst_s, a2t_s):
    c = pl.program_id(0)

    @pl.when(c == 0)
    def _():
        hidt16 = _bf(hid_ref[:].T)
        t_idx = jax.lax.broadcasted_iota(jnp.int32, (2 * K, K * N), 0)
        k_idx = jax.lax.broadcasted_iota(jnp.int32, (2 * K, K * N), 1) // N
        lhst_s[:2 * K, :] = (t_idx % K == k_idx).astype(jnp.float32).astype(jnp.bfloat16)
        for k in range(K):
            lhst_s[2 * K:, k * N:(k + 1) * N] = hidt16
        a = (jnp.dot(wf1a_ref[:], hidt16, preferred_element_type=jnp.float32)
             + bf1_ref[:])
        ahi = _bf(a)
        alo = _bf(a - ahi.astype(jnp.float32))
        for cc in range(NC):
            a2t_s[cc, :, :K] = ahi[:, cc * K:(cc + 1) * K]
            a2t_s[cc, :, K:] = alo[:, cc * K:(cc + 1) * K]
        s_ref[:] = jnp.zeros_like(s_ref)

    rhst = jnp.concatenate([a2t_s[c], wf1b_ref[:]], axis=1)
    pre = jnp.dot(rhst, lhst_s[:], preferred_element_type=jnp.float32)
    relu = jnp.maximum(pre, 0.0)
    msgs = jnp.dot(wf2_ref[:], relu, preferred_element_type=jnp.float32)
    m = adj3_ref[0].astype(jnp.float32)
    terms = [m[k:k + 1, :] * msgs[:, k * N:(k + 1) * N] for k in range(K)]
    while len(terms) > 1:
        terms = [terms[i] + terms[i + 1] for i in range(0, len(terms), 2)]
    s_ref[:] += terms[0]


def _tail_body(st_ref, adj_ref, x_ref, hprev_ref, cprev_ref,
               bf2_ref, wih_ref, whh_ref, bsum_ref,
               wo1_ref, bo1_ref, wo2_ref, bo2_ref,
               out_ref, h_ref, c_ref):
    deg = jnp.sum(adj_ref[:], axis=0, keepdims=True).astype(jnp.float32)
    msgt = st_ref[:] + deg * bf2_ref[:]

    inpt = jnp.concatenate([x_ref[:].T, msgt], axis=0).astype(jnp.bfloat16)
    gates = (jnp.dot(wih_ref[:], inpt, preferred_element_type=jnp.float32)
             + jnp.dot(whh_ref[:], _bf(hprev_ref[0].T),
                       preferred_element_type=jnp.float32)
             + bsum_ref[:])
    i_g = jax.nn.sigmoid(gates[0 * D:1 * D, :])
    f_g = jax.nn.sigmoid(gates[1 * D:2 * D, :])
    g_g = jnp.tanh(gates[2 * D:3 * D, :])
    o_g = jax.nn.sigmoid(gates[3 * D:4 * D, :])
    c_new = f_g * cprev_ref[0].T + i_g * g_g
    h_new = o_g * jnp.tanh(c_new)

    hid1 = jnp.maximum(
        jnp.dot(wo1_ref[:], h_new.astype(jnp.bfloat16),
                preferred_element_type=jnp.float32) + bo1_ref[:], 0.0)
    out = (jnp.dot(wo2_ref[:], hid1.astype(jnp.bfloat16),
                   preferred_element_type=jnp.float32) + bo2_ref[:])
    out_ref[:] = out.T
    h_ref[:] = h_new.T
    c_ref[:] = c_new.T


@functools.partial(jax.jit, static_argnames=("interpret",))
def _run(adjacency_matrix, x, hidden, h_h, h_c, Wf1, bf1, Wf2, bf2,
         W_ih, W_hh, b_ih, b_hh, Wo1, bo1, Wo2, bo2, interpret=False):
    n, d = hidden.shape
    adj3 = adjacency_matrix.reshape(NC, K, n)

    St = pl.pallas_call(
        _sum_body,
        grid=(NC,),
        in_specs=[
            pl.BlockSpec((n, d), lambda c: (0, 0)),
            pl.BlockSpec((F_HID, d), lambda c: (0, 0)),
            pl.BlockSpec((F_HID, d), lambda c: (0, 0)),
            pl.BlockSpec((F_HID, 1), lambda c: (0, 0)),
            pl.BlockSpec((MSG, F_HID), lambda c: (0, 0)),
            pl.BlockSpec((1, K, n), lambda c: (c, 0, 0)),
        ],
        out_specs=pl.BlockSpec((MSG, n), lambda c: (0, 0)),
        out_shape=jax.ShapeDtypeStruct((MSG, n), jnp.float32),
        scratch_shapes=[
            pltpu.VMEM((2 * K + d, K * n), jnp.bfloat16),
            pltpu.VMEM((NC, F_HID, 2 * K), jnp.bfloat16),
        ],
        interpret=interpret,
    )(hidden, _bf(Wf1[:, :d]), _bf(Wf1[:, d:]), bf1[:, None],
      _bf(Wf2).astype(jnp.float32), adj3)

    out, h_new, c_new = pl.pallas_call(
        _tail_body,
        out_shape=[
            jax.ShapeDtypeStruct((n, Wo2.shape[0]), jnp.float32),
            jax.ShapeDtypeStruct((n, d), jnp.float32),
            jax.ShapeDtypeStruct((n, d), jnp.float32),
        ],
        interpret=interpret,
    )(St, adjacency_matrix, x, h_h, h_c,
      bf2[:, None], _bf(W_ih), _bf(W_hh), (b_ih + b_hh)[:, None],
      _bf(Wo1), bo1[:, None], _bf(Wo2), bo2[:, None])
    return out, h_new, h_new[None, :, :], c_new[None, :, :]


def kernel(adjacency_matrix, x, hidden, h_h, h_c, Wf1, bf1, Wf2, bf2,
           W_ih, W_hh, b_ih, b_hh, Wo1, bo1, Wo2, bo2):
    return _run(adjacency_matrix, x, hidden, h_h, h_c, Wf1, bf1, Wf2, bf2,
                W_ih, W_hh, b_ih, b_hh, Wo1, bo1, Wo2, bo2)
